# baseline (speedup 1.0000x reference)
import functools

import jax
import jax.numpy as jnp
from jax import lax
from jax.experimental import pallas as pl
from jax.experimental.pallas import tpu as pltpu
from jax.experimental.pallas import tpu_sc as plsc

E = 8
TOP_K = 2
D_MODEL = 1024
D_FF = 2048
T = 2048

TK = T * TOP_K
B = 128
NB = TK // B + E
CAP = NB * B
NW = 32
GCHUNK = 32


def _setup_indices(router_logits):
    probs = jax.nn.softmax(router_logits, axis=-1)
    topk_w, topk_ids = lax.top_k(probs, TOP_K)
    topk_w = topk_w / jnp.sum(topk_w, axis=-1, keepdims=True)

    flat_e = topk_ids.reshape(-1).astype(jnp.int32)
    onehot = (flat_e[:, None] == jnp.arange(E, dtype=jnp.int32)[None, :])
    onehot = onehot.astype(jnp.int32)
    ranks_all = jnp.cumsum(onehot, axis=0) - onehot
    rank = jnp.take_along_axis(ranks_all, flat_e[:, None], axis=1)[:, 0]
    counts = jnp.sum(onehot, axis=0)
    nblk = (counts + B - 1) // B
    blk_cum = jnp.cumsum(nblk).astype(jnp.int32)
    blk_start = jnp.concatenate(
        [jnp.zeros((1,), jnp.int32), blk_cum[:-1]])
    slots = blk_start[flat_e] * B + rank

    used = blk_cum[-1]
    bidx = jnp.arange(NB, dtype=jnp.int32)
    be_raw = jnp.minimum(
        jnp.searchsorted(blk_cum, bidx, side="right"),
        E - 1).astype(jnp.int32)
    valid = (bidx < used).astype(jnp.int32)
    block_expert = jnp.where(valid == 1, be_raw, be_raw[used - 1])
    rowsel = jnp.where(valid == 1, bidx, used - 1).astype(jnp.int32)

    src_tok = (jnp.arange(CAP, dtype=jnp.int32) % T).at[slots].set(
        jnp.arange(TK, dtype=jnp.int32) // TOP_K)
    wslot = jnp.zeros((CAP,), jnp.float32).at[slots].set(topk_w.reshape(-1))
    wg = jnp.broadcast_to(wslot[:, None], (CAP, 128))
    return slots, src_tok, block_expert, rowsel, valid, wg


def _make_sc_gather(n_sub, d, dtype):
    mesh = plsc.VectorSubcoreMesh(core_axis_name="c", subcore_axis_name="s")
    rows_out = NW * n_sub * GCHUNK

    @functools.partial(
        pl.kernel, mesh=mesh,
        out_type=jax.ShapeDtypeStruct((rows_out, d), dtype),
        scratch_types=[
            pltpu.VMEM((n_sub, GCHUNK), jnp.int32),
            pltpu.VMEM((GCHUNK, d), dtype),
            pltpu.VMEM((GCHUNK, d), dtype),
            pltpu.SemaphoreType.DMA,
            pltpu.SemaphoreType.DMA,
        ],
    )
    def gather(table_hbm, idx_hbm, out_hbm, idx_v, rows0_v, rows1_v, s0, s1):
        wid = lax.axis_index("s") * 2 + lax.axis_index("c")
        pltpu.sync_copy(idx_hbm.at[wid], idx_v)
        base = wid * (n_sub * GCHUNK)
        bufs = (rows0_v, rows1_v)
        sems = (s0, s1)
        pending = {0: pltpu.async_copy(table_hbm.at[idx_v.at[0]],
                                       bufs[0], sems[0])}
        for s in range(n_sub):
            if s + 1 < n_sub:
                pending[s + 1] = pltpu.async_copy(
                    table_hbm.at[idx_v.at[s + 1]],
                    bufs[(s + 1) % 2], sems[(s + 1) % 2])
            pending.pop(s).wait()
            pltpu.sync_copy(bufs[s % 2],
                            out_hbm.at[pl.ds(base + s * GCHUNK, GCHUNK)])

    return gather


WQ = 4


def _moe_mm_kernel(be_ref, rs_ref, va_ref, xg_ref, w1_ref, w2_ref, wg_ref,
                   out_ref):
    @pl.when(va_ref[pl.program_id(0)] == 1)
    def _():
        xb = xg_ref[...].astype(jnp.bfloat16)
        q1 = (2 * D_FF) // WQ
        hs = []
        for q in range(WQ):
            w1q = w1_ref[0, q * q1:(q + 1) * q1, :].astype(jnp.bfloat16)
            hs.append(lax.dot_general(
                xb, w1q, (((1,), (1,)), ((), ())),
                preferred_element_type=jnp.float32))
        h = jnp.concatenate(hs, axis=1)
        g = h[:, :D_FF]
        u = h[:, D_FF:]
        act = (g * (1.0 / (1.0 + jnp.exp(-g))) * u).astype(jnp.bfloat16)
        q2 = D_FF // WQ
        y = jnp.zeros((B, D_MODEL), jnp.float32)
        for q in range(WQ):
            w2q = w2_ref[0, :, q * q2:(q + 1) * q2].astype(jnp.bfloat16)
            y = y + lax.dot_general(
                act[:, q * q2:(q + 1) * q2], w2q, (((1,), (1,)), ((), ())),
                preferred_element_type=jnp.float32)
        out_ref[...] = y * wg_ref[:, 0:1]


def _grouped_mm(block_expert, rowsel, valid, xg, w1b, w2b, wg,
                interpret=False):
    grid_spec = pltpu.PrefetchScalarGridSpec(
        num_scalar_prefetch=3,
        grid=(NB,),
        in_specs=[
            pl.BlockSpec((B, D_MODEL), lambda b, be, rs, va: (rs[b], 0)),
            pl.BlockSpec((1, 2 * D_FF, D_MODEL),
                         lambda b, be, rs, va: (be[b], 0, 0)),
            pl.BlockSpec((1, D_MODEL, D_FF),
                         lambda b, be, rs, va: (be[b], 0, 0)),
            pl.BlockSpec((B, 128), lambda b, be, rs, va: (rs[b], 0)),
        ],
        out_specs=pl.BlockSpec((B, D_MODEL), lambda b, be, rs, va: (rs[b], 0)),
    )
    return pl.pallas_call(
        _moe_mm_kernel,
        grid_spec=grid_spec,
        out_shape=jax.ShapeDtypeStruct((CAP, D_MODEL), jnp.float32),
        interpret=interpret,
    )(block_expert, rowsel, valid, xg, w1b, w2b, wg)


NT = T // NW
SC_N = 4
SCT = NT // SC_N


def _make_sc_combine():
    mesh = plsc.VectorSubcoreMesh(core_axis_name="c", subcore_axis_name="s")

    @functools.partial(
        pl.kernel, mesh=mesh,
        out_type=jax.ShapeDtypeStruct((T, D_MODEL), jnp.float32),
        scratch_types=[
            pltpu.VMEM((SC_N, SCT), jnp.int32),
            pltpu.VMEM((SC_N, SCT), jnp.int32),
            pltpu.VMEM((SCT, D_MODEL), jnp.float32),
            pltpu.VMEM((SCT, D_MODEL), jnp.float32),
            pltpu.VMEM((SCT, D_MODEL), jnp.float32),
            pltpu.VMEM((SCT, D_MODEL), jnp.float32),
            pltpu.SemaphoreType.DMA,
            pltpu.SemaphoreType.DMA,
            pltpu.SemaphoreType.DMA,
            pltpu.SemaphoreType.DMA,
        ],
    )
    def comb(yg_hbm, d0_hbm, d1_hbm, out_hbm,
             i0, i1, r0a, r1a, r0b, r1b, s0a, s1a, s0b, s1b):
        wid = lax.axis_index("s") * 2 + lax.axis_index("c")
        pltpu.sync_copy(d0_hbm.at[wid], i0)
        pltpu.sync_copy(d1_hbm.at[wid], i1)
        base = wid * NT
        r0 = (r0a, r0b)
        r1 = (r1a, r1b)
        s0 = (s0a, s0b)
        s1 = (s1a, s1b)
        pend = {0: (pltpu.async_copy(yg_hbm.at[i0.at[0]], r0[0], s0[0]),
                    pltpu.async_copy(yg_hbm.at[i1.at[0]], r1[0], s1[0]))}
        for s in range(SC_N):
            if s + 1 < SC_N:
                p = (s + 1) % 2
                pend[s + 1] = (
                    pltpu.async_copy(yg_hbm.at[i0.at[s + 1]], r0[p], s0[p]),
                    pltpu.async_copy(yg_hbm.at[i1.at[s + 1]], r1[p], s1[p]))
            c0, c1 = pend.pop(s)
            c0.wait()
            c1.wait()
            p = s % 2
            a, b = r0[p], r1[p]

            def body(i, carry, a=a, b=b):
                r = i >> 6
                sl = pl.ds((i & 63) * 16, 16)
                a[r, sl] = a[r, sl] + b[r, sl]
                return carry

            lax.fori_loop(0, SCT * 64, body, 0, unroll=8)
            pltpu.sync_copy(a, out_hbm.at[pl.ds(base + s * SCT, SCT)])

    return comb


def kernel(x, router_logits, w1, w2):
    (slots, src_tok, block_expert,
     rowsel, valid, wg) = _setup_indices(router_logits)

    gather_x = _make_sc_gather(CAP // NW // GCHUNK, D_MODEL, jnp.float32)
    xg = gather_x(x, src_tok.reshape(NW, CAP // NW // GCHUNK, GCHUNK))

    yg = _grouped_mm(block_expert, rowsel, valid, xg, w1, w2, wg)

    sr = slots.reshape(T, TOP_K)
    d0 = sr[:, 0].reshape(NW, SC_N, SCT)
    d1 = sr[:, 1].reshape(NW, SC_N, SCT)
    return _make_sc_combine()(yg, d0, d1)


# speedup vs baseline: 1.0836x; 1.0836x over previous
import functools

import jax
import jax.numpy as jnp
from jax import lax
from jax.experimental import pallas as pl
from jax.experimental.pallas import tpu as pltpu
from jax.experimental.pallas import tpu_sc as plsc

E = 8
TOP_K = 2
D_MODEL = 1024
D_FF = 2048
T = 2048

TK = T * TOP_K
B = 512
NB = TK // B + E
CAP = NB * B
NW = 32
GCHUNK = 32


def _setup_indices(router_logits):
    probs = jax.nn.softmax(router_logits, axis=-1)
    topk_w, topk_ids = lax.top_k(probs, TOP_K)
    topk_w = topk_w / jnp.sum(topk_w, axis=-1, keepdims=True)

    flat_e = topk_ids.reshape(-1).astype(jnp.int32)
    onehot = (flat_e[:, None] == jnp.arange(E, dtype=jnp.int32)[None, :])
    onehot = onehot.astype(jnp.int32)
    ranks_all = jnp.cumsum(onehot, axis=0) - onehot
    rank = jnp.take_along_axis(ranks_all, flat_e[:, None], axis=1)[:, 0]
    counts = jnp.sum(onehot, axis=0)
    nblk = (counts + B - 1) // B
    blk_cum = jnp.cumsum(nblk).astype(jnp.int32)
    blk_start = jnp.concatenate(
        [jnp.zeros((1,), jnp.int32), blk_cum[:-1]])
    slots = blk_start[flat_e] * B + rank

    used = blk_cum[-1]
    bidx = jnp.arange(NB, dtype=jnp.int32)
    be_raw = jnp.minimum(
        jnp.searchsorted(blk_cum, bidx, side="right"),
        E - 1).astype(jnp.int32)
    valid = (bidx < used).astype(jnp.int32)
    block_expert = jnp.where(valid == 1, be_raw, be_raw[used - 1])
    rowsel = jnp.where(valid == 1, bidx, used - 1).astype(jnp.int32)

    src_tok = (jnp.arange(CAP, dtype=jnp.int32) % T).at[slots].set(
        jnp.arange(TK, dtype=jnp.int32) // TOP_K)
    wslot = jnp.zeros((CAP,), jnp.float32).at[slots].set(topk_w.reshape(-1))
    wg = jnp.broadcast_to(wslot[:, None], (CAP, 128))
    return slots, src_tok, block_expert, rowsel, valid, wg


def _make_sc_gather(n_sub, d, dtype):
    mesh = plsc.VectorSubcoreMesh(core_axis_name="c", subcore_axis_name="s")
    rows_out = NW * n_sub * GCHUNK

    @functools.partial(
        pl.kernel, mesh=mesh,
        out_type=jax.ShapeDtypeStruct((rows_out, d), dtype),
        scratch_types=[
            pltpu.VMEM((n_sub, GCHUNK), jnp.int32),
            pltpu.VMEM((GCHUNK, d), dtype),
            pltpu.VMEM((GCHUNK, d), dtype),
            pltpu.SemaphoreType.DMA,
            pltpu.SemaphoreType.DMA,
        ],
    )
    def gather(table_hbm, idx_hbm, out_hbm, idx_v, rows0_v, rows1_v, s0, s1):
        wid = lax.axis_index("s") * 2 + lax.axis_index("c")
        pltpu.sync_copy(idx_hbm.at[wid], idx_v)
        base = wid * (n_sub * GCHUNK)
        bufs = (rows0_v, rows1_v)
        sems = (s0, s1)
        pending = {0: pltpu.async_copy(table_hbm.at[idx_v.at[0]],
                                       bufs[0], sems[0])}
        for s in range(n_sub):
            if s + 1 < n_sub:
                pending[s + 1] = pltpu.async_copy(
                    table_hbm.at[idx_v.at[s + 1]],
                    bufs[(s + 1) % 2], sems[(s + 1) % 2])
            pending.pop(s).wait()
            pltpu.sync_copy(bufs[s % 2],
                            out_hbm.at[pl.ds(base + s * GCHUNK, GCHUNK)])

    return gather


def _moe_mm_kernel(be_ref, rs_ref, va_ref, xg_ref, w1_ref, w2_ref, wg_ref,
                   out_ref):
    @pl.when(va_ref[pl.program_id(0)] == 1)
    def _():
        xb = xg_ref[...].astype(jnp.bfloat16)
        w1e = w1_ref[0]
        h = lax.dot_general(xb, w1e, (((1,), (1,)), ((), ())),
                            preferred_element_type=jnp.float32)
        g = h[:, :D_FF]
        u = h[:, D_FF:]
        act = (g * (1.0 / (1.0 + jnp.exp(-g))) * u).astype(jnp.bfloat16)
        w2e = w2_ref[0]
        y = lax.dot_general(act, w2e, (((1,), (1,)), ((), ())),
                            preferred_element_type=jnp.float32)
        out_ref[...] = y * wg_ref[:, 0:1]


def _grouped_mm(block_expert, rowsel, valid, xg, w1b, w2b, wg,
                interpret=False):
    grid_spec = pltpu.PrefetchScalarGridSpec(
        num_scalar_prefetch=3,
        grid=(NB,),
        in_specs=[
            pl.BlockSpec((B, D_MODEL), lambda b, be, rs, va: (rs[b], 0)),
            pl.BlockSpec((1, 2 * D_FF, D_MODEL),
                         lambda b, be, rs, va: (be[b], 0, 0)),
            pl.BlockSpec((1, D_MODEL, D_FF),
                         lambda b, be, rs, va: (be[b], 0, 0)),
            pl.BlockSpec((B, 128), lambda b, be, rs, va: (rs[b], 0)),
        ],
        out_specs=pl.BlockSpec((B, D_MODEL), lambda b, be, rs, va: (rs[b], 0)),
    )
    return pl.pallas_call(
        _moe_mm_kernel,
        grid_spec=grid_spec,
        out_shape=jax.ShapeDtypeStruct((CAP, D_MODEL), jnp.float32),
        interpret=interpret,
    )(block_expert, rowsel, valid, xg, w1b, w2b, wg)


NT = T // NW
SC_N = 4
SCT = NT // SC_N


def _make_sc_combine():
    mesh = plsc.VectorSubcoreMesh(core_axis_name="c", subcore_axis_name="s")

    @functools.partial(
        pl.kernel, mesh=mesh,
        out_type=jax.ShapeDtypeStruct((T, D_MODEL), jnp.float32),
        scratch_types=[
            pltpu.VMEM((SC_N, SCT), jnp.int32),
            pltpu.VMEM((SC_N, SCT), jnp.int32),
            pltpu.VMEM((SCT, D_MODEL), jnp.float32),
            pltpu.VMEM((SCT, D_MODEL), jnp.float32),
            pltpu.VMEM((SCT, D_MODEL), jnp.float32),
            pltpu.VMEM((SCT, D_MODEL), jnp.float32),
            pltpu.SemaphoreType.DMA,
            pltpu.SemaphoreType.DMA,
            pltpu.SemaphoreType.DMA,
            pltpu.SemaphoreType.DMA,
        ],
    )
    def comb(yg_hbm, d0_hbm, d1_hbm, out_hbm,
             i0, i1, r0a, r1a, r0b, r1b, s0a, s1a, s0b, s1b):
        wid = lax.axis_index("s") * 2 + lax.axis_index("c")
        pltpu.sync_copy(d0_hbm.at[wid], i0)
        pltpu.sync_copy(d1_hbm.at[wid], i1)
        base = wid * NT
        r0 = (r0a, r0b)
        r1 = (r1a, r1b)
        s0 = (s0a, s0b)
        s1 = (s1a, s1b)
        pend = {0: (pltpu.async_copy(yg_hbm.at[i0.at[0]], r0[0], s0[0]),
                    pltpu.async_copy(yg_hbm.at[i1.at[0]], r1[0], s1[0]))}
        for s in range(SC_N):
            if s + 1 < SC_N:
                p = (s + 1) % 2
                pend[s + 1] = (
                    pltpu.async_copy(yg_hbm.at[i0.at[s + 1]], r0[p], s0[p]),
                    pltpu.async_copy(yg_hbm.at[i1.at[s + 1]], r1[p], s1[p]))
            c0, c1 = pend.pop(s)
            c0.wait()
            c1.wait()
            p = s % 2
            a, b = r0[p], r1[p]

            def body(i, carry, a=a, b=b):
                r = i >> 6
                sl = pl.ds((i & 63) * 16, 16)
                a[r, sl] = a[r, sl] + b[r, sl]
                return carry

            lax.fori_loop(0, SCT * 64, body, 0, unroll=8)
            pltpu.sync_copy(a, out_hbm.at[pl.ds(base + s * SCT, SCT)])

    return comb


def kernel(x, router_logits, w1, w2):
    (slots, src_tok, block_expert,
     rowsel, valid, wg) = _setup_indices(router_logits)

    gather_x = _make_sc_gather(CAP // NW // GCHUNK, D_MODEL, jnp.float32)
    xg = gather_x(x, src_tok.reshape(NW, CAP // NW // GCHUNK, GCHUNK))

    yg = _grouped_mm(block_expert, rowsel, valid, xg,
                     w1.astype(jnp.bfloat16), w2.astype(jnp.bfloat16), wg)

    sr = slots.reshape(T, TOP_K)
    d0 = sr[:, 0].reshape(NW, SC_N, SCT)
    d1 = sr[:, 1].reshape(NW, SC_N, SCT)
    return _make_sc_combine()(yg, d0, d1)


# speedup vs baseline: 1.1976x; 1.1052x over previous
import functools

import jax
import jax.numpy as jnp
from jax import lax
from jax.experimental import pallas as pl
from jax.experimental.pallas import tpu as pltpu
from jax.experimental.pallas import tpu_sc as plsc

E = 8
TOP_K = 2
D_MODEL = 1024
D_FF = 2048
T = 2048

TK = T * TOP_K
B = 256
NB = TK // B + E
CAP = NB * B
NW = 32
GCHUNK = 32


def _setup_indices(router_logits):
    probs = jax.nn.softmax(router_logits, axis=-1)
    topk_w, topk_ids = lax.top_k(probs, TOP_K)
    topk_w = topk_w / jnp.sum(topk_w, axis=-1, keepdims=True)

    flat_e = topk_ids.reshape(-1).astype(jnp.int32)
    onehot = (flat_e[:, None] == jnp.arange(E, dtype=jnp.int32)[None, :])
    onehot = onehot.astype(jnp.int32)
    ranks_all = jnp.cumsum(onehot, axis=0) - onehot
    rank = jnp.take_along_axis(ranks_all, flat_e[:, None], axis=1)[:, 0]
    counts = jnp.sum(onehot, axis=0)
    nblk = (counts + B - 1) // B
    blk_cum = jnp.cumsum(nblk).astype(jnp.int32)
    blk_start = jnp.concatenate(
        [jnp.zeros((1,), jnp.int32), blk_cum[:-1]])
    slots = blk_start[flat_e] * B + rank

    used = blk_cum[-1]
    bidx = jnp.arange(NB, dtype=jnp.int32)
    be_raw = jnp.minimum(
        jnp.searchsorted(blk_cum, bidx, side="right"),
        E - 1).astype(jnp.int32)
    valid = (bidx < used).astype(jnp.int32)
    block_expert = jnp.where(valid == 1, be_raw, be_raw[used - 1])
    rowsel = jnp.where(valid == 1, bidx, used - 1).astype(jnp.int32)

    src_tok = (jnp.arange(CAP, dtype=jnp.int32) % T).at[slots].set(
        jnp.arange(TK, dtype=jnp.int32) // TOP_K)
    wslot = jnp.zeros((CAP,), jnp.float32).at[slots].set(topk_w.reshape(-1))
    wg = jnp.broadcast_to(wslot[:, None], (CAP, 128))
    return slots, src_tok, block_expert, rowsel, valid, wg


def _make_sc_gather(n_sub, d, dtype):
    mesh = plsc.VectorSubcoreMesh(core_axis_name="c", subcore_axis_name="s")
    rows_out = NW * n_sub * GCHUNK

    @functools.partial(
        pl.kernel, mesh=mesh,
        out_type=jax.ShapeDtypeStruct((rows_out, d), dtype),
        scratch_types=[
            pltpu.VMEM((n_sub, GCHUNK), jnp.int32),
            pltpu.VMEM((GCHUNK, d), dtype),
            pltpu.VMEM((GCHUNK, d), dtype),
            pltpu.SemaphoreType.DMA,
            pltpu.SemaphoreType.DMA,
        ],
    )
    def gather(table_hbm, idx_hbm, out_hbm, idx_v, rows0_v, rows1_v, s0, s1):
        wid = lax.axis_index("s") * 2 + lax.axis_index("c")
        pltpu.sync_copy(idx_hbm.at[wid], idx_v)
        base = wid * (n_sub * GCHUNK)
        bufs = (rows0_v, rows1_v)
        sems = (s0, s1)
        pending = {0: pltpu.async_copy(table_hbm.at[idx_v.at[0]],
                                       bufs[0], sems[0])}
        for s in range(n_sub):
            if s + 1 < n_sub:
                pending[s + 1] = pltpu.async_copy(
                    table_hbm.at[idx_v.at[s + 1]],
                    bufs[(s + 1) % 2], sems[(s + 1) % 2])
            pending.pop(s).wait()
            pltpu.sync_copy(bufs[s % 2],
                            out_hbm.at[pl.ds(base + s * GCHUNK, GCHUNK)])

    return gather


def _moe_mm_kernel(be_ref, rs_ref, va_ref, xg_ref, w1_ref, w2_ref, wg_ref,
                   out_ref):
    @pl.when(va_ref[pl.program_id(0)] == 1)
    def _():
        xb = xg_ref[...].astype(jnp.bfloat16)
        w1e = w1_ref[0]
        h = lax.dot_general(xb, w1e, (((1,), (1,)), ((), ())),
                            preferred_element_type=jnp.float32)
        g = h[:, :D_FF]
        u = h[:, D_FF:]
        act = (g * (1.0 / (1.0 + jnp.exp(-g))) * u).astype(jnp.bfloat16)
        w2e = w2_ref[0].astype(jnp.bfloat16)
        y = lax.dot_general(act, w2e, (((1,), (1,)), ((), ())),
                            preferred_element_type=jnp.float32)
        out_ref[...] = y * wg_ref[:, 0:1]


def _grouped_mm(block_expert, rowsel, valid, xg, w1b, w2b, wg,
                interpret=False):
    grid_spec = pltpu.PrefetchScalarGridSpec(
        num_scalar_prefetch=3,
        grid=(NB,),
        in_specs=[
            pl.BlockSpec((B, D_MODEL), lambda b, be, rs, va: (rs[b], 0)),
            pl.BlockSpec((1, 2 * D_FF, D_MODEL),
                         lambda b, be, rs, va: (be[b], 0, 0)),
            pl.BlockSpec((1, D_MODEL, D_FF),
                         lambda b, be, rs, va: (be[b], 0, 0)),
            pl.BlockSpec((B, 128), lambda b, be, rs, va: (rs[b], 0)),
        ],
        out_specs=pl.BlockSpec((B, D_MODEL), lambda b, be, rs, va: (rs[b], 0)),
    )
    return pl.pallas_call(
        _moe_mm_kernel,
        grid_spec=grid_spec,
        out_shape=jax.ShapeDtypeStruct((CAP, D_MODEL), jnp.float32),
        interpret=interpret,
    )(block_expert, rowsel, valid, xg, w1b, w2b, wg)


NT = T // NW
SC_N = 4
SCT = NT // SC_N


def _make_sc_combine():
    mesh = plsc.VectorSubcoreMesh(core_axis_name="c", subcore_axis_name="s")

    @functools.partial(
        pl.kernel, mesh=mesh,
        out_type=jax.ShapeDtypeStruct((T, D_MODEL), jnp.float32),
        scratch_types=[
            pltpu.VMEM((SC_N, SCT), jnp.int32),
            pltpu.VMEM((SC_N, SCT), jnp.int32),
            pltpu.VMEM((SCT, D_MODEL), jnp.float32),
            pltpu.VMEM((SCT, D_MODEL), jnp.float32),
            pltpu.VMEM((SCT, D_MODEL), jnp.float32),
            pltpu.VMEM((SCT, D_MODEL), jnp.float32),
            pltpu.SemaphoreType.DMA,
            pltpu.SemaphoreType.DMA,
            pltpu.SemaphoreType.DMA,
            pltpu.SemaphoreType.DMA,
        ],
    )
    def comb(yg_hbm, d0_hbm, d1_hbm, out_hbm,
             i0, i1, r0a, r1a, r0b, r1b, s0a, s1a, s0b, s1b):
        wid = lax.axis_index("s") * 2 + lax.axis_index("c")
        pltpu.sync_copy(d0_hbm.at[wid], i0)
        pltpu.sync_copy(d1_hbm.at[wid], i1)
        base = wid * NT
        r0 = (r0a, r0b)
        r1 = (r1a, r1b)
        s0 = (s0a, s0b)
        s1 = (s1a, s1b)
        pend = {0: (pltpu.async_copy(yg_hbm.at[i0.at[0]], r0[0], s0[0]),
                    pltpu.async_copy(yg_hbm.at[i1.at[0]], r1[0], s1[0]))}
        for s in range(SC_N):
            if s + 1 < SC_N:
                p = (s + 1) % 2
                pend[s + 1] = (
                    pltpu.async_copy(yg_hbm.at[i0.at[s + 1]], r0[p], s0[p]),
                    pltpu.async_copy(yg_hbm.at[i1.at[s + 1]], r1[p], s1[p]))
            c0, c1 = pend.pop(s)
            c0.wait()
            c1.wait()
            p = s % 2
            a, b = r0[p], r1[p]

            def body(i, carry, a=a, b=b):
                r = i >> 6
                sl = pl.ds((i & 63) * 16, 16)
                a[r, sl] = a[r, sl] + b[r, sl]
                return carry

            lax.fori_loop(0, SCT * 64, body, 0, unroll=8)
            pltpu.sync_copy(a, out_hbm.at[pl.ds(base + s * SCT, SCT)])

    return comb


def kernel(x, router_logits, w1, w2):
    (slots, src_tok, block_expert,
     rowsel, valid, wg) = _setup_indices(router_logits)

    gather_x = _make_sc_gather(CAP // NW // GCHUNK, D_MODEL, jnp.float32)
    xg = gather_x(x, src_tok.reshape(NW, CAP // NW // GCHUNK, GCHUNK))

    yg = _grouped_mm(block_expert, rowsel, valid, xg,
                     w1.astype(jnp.bfloat16), w2, wg)

    sr = slots.reshape(T, TOP_K)
    d0 = sr[:, 0].reshape(NW, SC_N, SCT)
    d1 = sr[:, 1].reshape(NW, SC_N, SCT)
    return _make_sc_combine()(yg, d0, d1)


# speedup vs baseline: 1.2085x; 1.0091x over previous
import functools

import jax
import jax.numpy as jnp
from jax import lax
from jax.experimental import pallas as pl
from jax.experimental.pallas import tpu as pltpu
from jax.experimental.pallas import tpu_sc as plsc

E = 8
TOP_K = 2
D_MODEL = 1024
D_FF = 2048
T = 2048

TK = T * TOP_K
B = 512
NB = TK // B + E
CAP = NB * B
NW = 32
GCHUNK = 32


def _setup_indices(router_logits):
    probs = jax.nn.softmax(router_logits, axis=-1)
    topk_w, topk_ids = lax.top_k(probs, TOP_K)
    topk_w = topk_w / jnp.sum(topk_w, axis=-1, keepdims=True)

    flat_e = topk_ids.reshape(-1).astype(jnp.int32)
    onehot = (flat_e[:, None] == jnp.arange(E, dtype=jnp.int32)[None, :])
    onehot = onehot.astype(jnp.int32)
    ranks_all = jnp.cumsum(onehot, axis=0) - onehot
    rank = jnp.take_along_axis(ranks_all, flat_e[:, None], axis=1)[:, 0]
    counts = jnp.sum(onehot, axis=0)
    nblk = (counts + B - 1) // B
    blk_cum = jnp.cumsum(nblk).astype(jnp.int32)
    blk_start = jnp.concatenate(
        [jnp.zeros((1,), jnp.int32), blk_cum[:-1]])
    slots = blk_start[flat_e] * B + rank

    used = blk_cum[-1]
    bidx = jnp.arange(NB, dtype=jnp.int32)
    be_raw = jnp.minimum(
        jnp.searchsorted(blk_cum, bidx, side="right"),
        E - 1).astype(jnp.int32)
    valid = (bidx < used).astype(jnp.int32)
    block_expert = jnp.where(valid == 1, be_raw, be_raw[used - 1])
    rowsel = jnp.where(valid == 1, bidx, used - 1).astype(jnp.int32)

    src_tok = (jnp.arange(CAP, dtype=jnp.int32) % T).at[slots].set(
        jnp.arange(TK, dtype=jnp.int32) // TOP_K)
    wslot = jnp.zeros((CAP,), jnp.float32).at[slots].set(topk_w.reshape(-1))
    wg = jnp.broadcast_to(wslot[:, None], (CAP, 128))
    return slots, src_tok, block_expert, rowsel, valid, wg


def _make_sc_gather(n_sub, d, dtype):
    mesh = plsc.VectorSubcoreMesh(core_axis_name="c", subcore_axis_name="s")
    rows_out = NW * n_sub * GCHUNK

    @functools.partial(
        pl.kernel, mesh=mesh,
        out_type=jax.ShapeDtypeStruct((rows_out, d), dtype),
        scratch_types=[
            pltpu.VMEM((n_sub, GCHUNK), jnp.int32),
            pltpu.VMEM((GCHUNK, d), dtype),
            pltpu.VMEM((GCHUNK, d), dtype),
            pltpu.SemaphoreType.DMA,
            pltpu.SemaphoreType.DMA,
        ],
    )
    def gather(table_hbm, idx_hbm, out_hbm, idx_v, rows0_v, rows1_v, s0, s1):
        wid = lax.axis_index("s") * 2 + lax.axis_index("c")
        pltpu.sync_copy(idx_hbm.at[wid], idx_v)
        base = wid * (n_sub * GCHUNK)
        bufs = (rows0_v, rows1_v)
        sems = (s0, s1)
        pending = {0: pltpu.async_copy(table_hbm.at[idx_v.at[0]],
                                       bufs[0], sems[0])}
        for s in range(n_sub):
            if s + 1 < n_sub:
                pending[s + 1] = pltpu.async_copy(
                    table_hbm.at[idx_v.at[s + 1]],
                    bufs[(s + 1) % 2], sems[(s + 1) % 2])
            pending.pop(s).wait()
            pltpu.sync_copy(bufs[s % 2],
                            out_hbm.at[pl.ds(base + s * GCHUNK, GCHUNK)])

    return gather


def _moe_mm_kernel(be_ref, rs_ref, va_ref, xg_ref, w1_ref, w2_ref, wg_ref,
                   out_ref):
    @pl.when(va_ref[pl.program_id(0)] == 1)
    def _():
        xb = xg_ref[...].astype(jnp.bfloat16)
        w1e = w1_ref[0]
        h = lax.dot_general(xb, w1e, (((1,), (1,)), ((), ())),
                            preferred_element_type=jnp.float32)
        g = h[:, :D_FF]
        u = h[:, D_FF:]
        act = (g * (1.0 / (1.0 + jnp.exp(-g))) * u).astype(jnp.bfloat16)
        w2e = w2_ref[0].astype(jnp.bfloat16)
        y = lax.dot_general(act, w2e, (((1,), (1,)), ((), ())),
                            preferred_element_type=jnp.float32)
        out_ref[...] = y * wg_ref[:, 0:1]


def _grouped_mm(block_expert, rowsel, valid, xg, w1b, w2b, wg,
                interpret=False):
    grid_spec = pltpu.PrefetchScalarGridSpec(
        num_scalar_prefetch=3,
        grid=(NB,),
        in_specs=[
            pl.BlockSpec((B, D_MODEL), lambda b, be, rs, va: (rs[b], 0)),
            pl.BlockSpec((1, 2 * D_FF, D_MODEL),
                         lambda b, be, rs, va: (be[b], 0, 0)),
            pl.BlockSpec((1, D_MODEL, D_FF),
                         lambda b, be, rs, va: (be[b], 0, 0)),
            pl.BlockSpec((B, 128), lambda b, be, rs, va: (rs[b], 0)),
        ],
        out_specs=pl.BlockSpec((B, D_MODEL), lambda b, be, rs, va: (rs[b], 0)),
    )
    return pl.pallas_call(
        _moe_mm_kernel,
        grid_spec=grid_spec,
        out_shape=jax.ShapeDtypeStruct((CAP, D_MODEL), jnp.float32),
        interpret=interpret,
    )(block_expert, rowsel, valid, xg, w1b, w2b, wg)


NT = T // NW
SC_N = 4
SCT = NT // SC_N


def _make_sc_combine():
    mesh = plsc.VectorSubcoreMesh(core_axis_name="c", subcore_axis_name="s")

    @functools.partial(
        pl.kernel, mesh=mesh,
        out_type=jax.ShapeDtypeStruct((T, D_MODEL), jnp.float32),
        scratch_types=[
            pltpu.VMEM((SC_N, SCT), jnp.int32),
            pltpu.VMEM((SC_N, SCT), jnp.int32),
            pltpu.VMEM((SCT, D_MODEL), jnp.float32),
            pltpu.VMEM((SCT, D_MODEL), jnp.float32),
            pltpu.VMEM((SCT, D_MODEL), jnp.float32),
            pltpu.VMEM((SCT, D_MODEL), jnp.float32),
            pltpu.SemaphoreType.DMA,
            pltpu.SemaphoreType.DMA,
            pltpu.SemaphoreType.DMA,
            pltpu.SemaphoreType.DMA,
        ],
    )
    def comb(yg_hbm, d0_hbm, d1_hbm, out_hbm,
             i0, i1, r0a, r1a, r0b, r1b, s0a, s1a, s0b, s1b):
        wid = lax.axis_index("s") * 2 + lax.axis_index("c")
        pltpu.sync_copy(d0_hbm.at[wid], i0)
        pltpu.sync_copy(d1_hbm.at[wid], i1)
        base = wid * NT
        r0 = (r0a, r0b)
        r1 = (r1a, r1b)
        s0 = (s0a, s0b)
        s1 = (s1a, s1b)
        pend = {0: (pltpu.async_copy(yg_hbm.at[i0.at[0]], r0[0], s0[0]),
                    pltpu.async_copy(yg_hbm.at[i1.at[0]], r1[0], s1[0]))}
        for s in range(SC_N):
            if s + 1 < SC_N:
                p = (s + 1) % 2
                pend[s + 1] = (
                    pltpu.async_copy(yg_hbm.at[i0.at[s + 1]], r0[p], s0[p]),
                    pltpu.async_copy(yg_hbm.at[i1.at[s + 1]], r1[p], s1[p]))
            c0, c1 = pend.pop(s)
            c0.wait()
            c1.wait()
            p = s % 2
            a, b = r0[p], r1[p]

            def body(i, carry, a=a, b=b):
                r = i >> 6
                sl = pl.ds((i & 63) * 16, 16)
                a[r, sl] = a[r, sl] + b[r, sl]
                return carry

            lax.fori_loop(0, SCT * 64, body, 0, unroll=8)
            pltpu.sync_copy(a, out_hbm.at[pl.ds(base + s * SCT, SCT)])

    return comb


def kernel(x, router_logits, w1, w2):
    (slots, src_tok, block_expert,
     rowsel, valid, wg) = _setup_indices(router_logits)

    gather_x = _make_sc_gather(CAP // NW // GCHUNK, D_MODEL, jnp.float32)
    xg = gather_x(x, src_tok.reshape(NW, CAP // NW // GCHUNK, GCHUNK))

    yg = _grouped_mm(block_expert, rowsel, valid, xg,
                     w1.astype(jnp.bfloat16), w2, wg)

    sr = slots.reshape(T, TOP_K)
    d0 = sr[:, 0].reshape(NW, SC_N, SCT)
    d1 = sr[:, 1].reshape(NW, SC_N, SCT)
    return _make_sc_combine()(yg, d0, d1)
